# Initial kernel scaffold; baseline (speedup 1.0000x reference)
import jax
import jax.numpy as jnp
from jax.experimental import pallas as pl


def kernel(score, target):
    raise NotImplementedError("write your pallas kernel here")


# speedup vs baseline: 5.4916x; 5.4916x over previous
import functools

import jax
import jax.numpy as jnp
from jax import lax
from jax.experimental import pallas as pl
from jax.experimental.pallas import tpu as pltpu

_BITS_07 = 0x3F333333


def _ohem_body(score_t_ref, tgt_ref, out_ref, loss_s, pred_s, *, nb, kth):
    i = pl.program_id(0)

    @pl.when(i < nb)
    def _dense():
        x = score_t_ref[...]
        c, bl = x.shape
        t = tgt_ref[0]
        cls = lax.broadcasted_iota(jnp.int32, (c, bl), 0)
        e = jnp.exp(x)
        s = jnp.sum(e, axis=0, keepdims=True)
        xt = jnp.sum(jnp.where(cls == t, x, 0.0), axis=0, keepdims=True)
        loss_s[pl.ds(i, 1), :] = jnp.log(s) - xt
        pred_s[pl.ds(i, 1), :] = jnp.exp(xt) / s

    @pl.when(i == nb)
    def _select():
        bits = lax.bitcast_convert_type(pred_s[...], jnp.int32)
        losses = loss_s[...]

        def bs_body(_, carry):
            lo, hi = carry
            mid = lax.div(lo + hi, 2)
            cnt = jnp.sum((bits <= mid).astype(jnp.int32))
            geq = cnt >= kth + 1
            return (jnp.where(geq, lo, mid + 1), jnp.where(geq, mid, hi))

        lo, _ = lax.fori_loop(0, 31, bs_body,
                              (jnp.int32(0), jnp.int32(1 << 30)))
        thr = jnp.maximum(lo, _BITS_07)
        keep = bits < thr
        num = jnp.sum(jnp.where(keep, losses, 0.0))
        den = jnp.sum(keep.astype(jnp.float32))
        out_ref[...] = (num / den)[None, None]


def kernel(score, target):
    n, c = score.shape
    bl = 2048
    nb = n // bl
    kth = min(int(0.7 * n), n - 1)
    score_t = score.T
    tgt3 = target.reshape(nb, 1, bl)
    out = pl.pallas_call(
        functools.partial(_ohem_body, nb=nb, kth=kth),
        grid=(nb + 1,),
        in_specs=[
            pl.BlockSpec((c, bl), lambda i: (0, jnp.minimum(i, nb - 1))),
            pl.BlockSpec((1, 1, bl), lambda i: (jnp.minimum(i, nb - 1), 0, 0)),
        ],
        out_specs=pl.BlockSpec((1, 1), lambda i: (0, 0)),
        out_shape=jax.ShapeDtypeStruct((1, 1), jnp.float32),
        scratch_shapes=[
            pltpu.VMEM((nb, bl), jnp.float32),
            pltpu.VMEM((nb, bl), jnp.float32),
        ],
    )(score_t, tgt3)
    return out[0, 0]
